# baseline (speedup 1.0000x reference)
import jax
import jax.numpy as jnp
from jax.experimental import pallas as pl
from jax.experimental.pallas import tpu as pltpu

_E = 3200000
_BLK = 128000


def _copy2d_body(x_ref, o_ref):
    o_ref[...] = x_ref[...]


def _copy1d_body(x_ref, o_ref):
    o_ref[...] = x_ref[...]


def _copy2d(x):
    return pl.pallas_call(
        _copy2d_body,
        grid=(_E // _BLK,),
        in_specs=[pl.BlockSpec((2, _BLK), lambda i: (jnp.int32(0), i))],
        out_specs=pl.BlockSpec((2, _BLK), lambda i: (jnp.int32(0), i)),
        out_shape=jax.ShapeDtypeStruct(x.shape, x.dtype),
    )(x)


def _copy1d(x):
    return pl.pallas_call(
        _copy1d_body,
        grid=(_E // _BLK,),
        in_specs=[pl.BlockSpec((_BLK,), lambda i: (i,))],
        out_specs=pl.BlockSpec((_BLK,), lambda i: (i,)),
        out_shape=jax.ShapeDtypeStruct(x.shape, x.dtype),
    )(x)


def kernel(edgeparam, subjparam, objparam, edge_index, edge_type):
    ei_dtype, et_dtype = edge_index.dtype, edge_type.dtype
    wide = jnp.dtype(ei_dtype).itemsize == 8
    ei_in = edge_index.astype(jnp.uint32) if wide else edge_index
    et_in = edge_type.astype(jnp.uint32) if wide else edge_type

    ei_out = _copy2d(ei_in)
    et_out = _copy1d(et_in)

    if wide:
        ei_out = ei_out.astype(jnp.uint64).astype(ei_dtype)
        et_out = et_out.astype(jnp.uint64).astype(et_dtype)
    return (ei_out, et_out)
